# baseline (speedup 1.0000x reference)
import functools

import jax
import jax.numpy as jnp
from jax import lax
from jax.experimental import pallas as pl
from jax.experimental.pallas import tpu as pltpu
from jax.experimental.pallas import tpu_sc as plsc

N = 10000
E = 320000
CH = 128
NCHUNK = E // CH
NW = 32
CPW = NCHUNK // NW
XTRA = NCHUNK - CPW * NW
GS = 6
NG = CPW // GS
RPS = 624
REM = N - 16 * RPS

_SC_PARAMS = pltpu.CompilerParams(use_tc_tiling_on_sc=False)


def _zero_rows(ref, nrows, ncols):
    z = jnp.zeros((16,), jnp.float32)

    def body(i, carry):
        for k in range(ncols // 16):
            ref[i, pl.ds(k * 16, 16)] = z
        return carry

    lax.fori_loop(0, nrows, body, 0)


def _zero_acc_slice(zbuf, acc_sh, s):
    row0 = s * RPS
    for k in range(4):
        pltpu.sync_copy(zbuf, acc_sh.at[pl.ds(row0 + k * CH, CH)])
    pltpu.sync_copy(zbuf.at[pl.ds(0, RPS - 4 * CH)],
                    acc_sh.at[pl.ds(row0 + 4 * CH, RPS - 4 * CH)])

    @pl.when(s == 15)
    def _():
        pltpu.sync_copy(zbuf.at[pl.ds(0, REM)],
                        acc_sh.at[pl.ds(16 * RPS, REM)])


def _copy_out(acc_sh, out_hbm, c, s):
    row0 = s * RPS
    pltpu.sync_copy(acc_sh.at[pl.ds(row0, RPS)],
                    out_hbm.at[c].at[pl.ds(row0, RPS)])

    @pl.when(s == 15)
    def _():
        pltpu.sync_copy(acc_sh.at[pl.ds(16 * RPS, REM)],
                        out_hbm.at[c].at[pl.ds(16 * RPS, REM)])


def _make_agg(F, NBUF):
    mesh = plsc.VectorSubcoreMesh(core_axis_name="c", subcore_axis_name="s")

    @functools.partial(
        pl.kernel,
        out_type=jax.ShapeDtypeStruct((2, N, F), jnp.float32),
        mesh=mesh,
        compiler_params=_SC_PARAMS,
        scratch_types=(
            [pltpu.VMEM((GS, CH), jnp.int32)] * 4
            + [pltpu.VMEM((CH, F), jnp.float32)] * NBUF
            + [pltpu.VMEM_SHARED((N, F), jnp.float32)]
            + [pltpu.SemaphoreType.DMA] * (2 * NBUF)
        ),
    )
    def agg(g_hbm, ei2_hbm, out_hbm, *scr):
        sbufs = scr[0:2]
        dbufs = scr[2:4]
        rows = scr[4:4 + NBUF]
        acc_sh = scr[4 + NBUF]
        gsems = scr[5 + NBUF:5 + 2 * NBUF]
        ssems = scr[5 + 2 * NBUF:5 + 3 * NBUF]
        c = lax.axis_index("c")
        s = lax.axis_index("s")
        wid = s * 2 + c
        cstart = wid * CPW + jnp.minimum(wid, XTRA)
        extra = wid < XTRA

        for r in rows:
            _zero_rows(r, CH, F)
        _zero_acc_slice(rows[0], acc_sh, s)
        plsc.subcore_barrier()

        def load_group(g, p):
            pltpu.sync_copy(ei2_hbm.at[0].at[pl.ds(cstart + g * GS, GS)],
                            sbufs[p])
            pltpu.sync_copy(ei2_hbm.at[1].at[pl.ds(cstart + g * GS, GS)],
                            dbufs[p])

        def gather(p, t, b):
            return pltpu.async_copy(
                g_hbm.at[sbufs[p].at[t]], rows[b], gsems[b])

        def wait_gather(b):
            pltpu.make_async_copy(g_hbm.at[sbufs[0].at[0]],
                                  rows[b], gsems[b]).wait()

        def scatter_start(p, t, b):
            pltpu.make_async_copy(rows[b], acc_sh.at[dbufs[p].at[t]],
                                  ssems[b]).start(add=True)

        def wait_scatter(b):
            pltpu.make_async_copy(rows[b], acc_sh.at[dbufs[0].at[0]],
                                  ssems[b]).wait()

        def process(p, t, o, nxt):
            b = o % NBUF
            nb = (o + 1) % NBUF
            wait_scatter(nb)
            if nxt is not None:
                gather(nxt[0], nxt[1], nb)
            wait_gather(b)
            scatter_start(p, t, b)

        def half_group(p, t0, o0, nxt_last):
            for t in range(t0, t0 + GS // 2):
                nxt = (p, t + 1) if t < GS - 1 else (nxt_last, 0)
                process(p, t, o0 + t - t0, nxt)

        load_group(0, 0)
        for b in range(1, NBUF):
            pltpu.make_async_copy(rows[b], acc_sh.at[dbufs[0].at[0]],
                                  ssems[b]).start(add=True)
        gather(0, 0, 0)

        def body(k, carry):
            half_group(0, 0, 0, None)
            load_group(2 * k + 1, 1)
            half_group(0, GS // 2, GS // 2, 1)
            half_group(1, 0, GS, None)
            load_group(2 * k + 2, 0)
            half_group(1, GS // 2, GS + GS // 2, 0)
            return carry

        lax.fori_loop(0, NG // 2, body, 0)

        for t in range(GS):
            process(0, t, t, (0, t + 1) if t < GS - 1 else None)
        starts = [CPW // NBUF + (1 if k < CPW % NBUF else 0) + (k >= 1)
                  for k in range(NBUF)]
        waits = [sum(1 for j in range(CPW) if (j + 1) % NBUF == k)
                 for k in range(NBUF)]
        for k in range(NBUF):
            for _ in range(starts[k] - waits[k]):
                wait_scatter(k)

        @pl.when(extra)
        def _():
            pltpu.sync_copy(ei2_hbm.at[0].at[pl.ds(cstart + CPW, 1)],
                            sbufs[1].at[pl.ds(0, 1)])
            pltpu.sync_copy(ei2_hbm.at[1].at[pl.ds(cstart + CPW, 1)],
                            dbufs[1].at[pl.ds(0, 1)])
            gather(1, 0, 0).wait()
            pltpu.sync_copy(rows[0], acc_sh.at[dbufs[1].at[0]], add=True)

        plsc.subcore_barrier()
        _copy_out(acc_sh, out_hbm, c, s)

    return agg


def _make_deg():
    mesh = plsc.VectorSubcoreMesh(core_axis_name="c", subcore_axis_name="s")

    @functools.partial(
        pl.kernel,
        out_type=jax.ShapeDtypeStruct((2, N, 16), jnp.float32),
        mesh=mesh,
        compiler_params=_SC_PARAMS,
        scratch_types=[
            pltpu.VMEM((CPW + 1, CH), jnp.int32),
            pltpu.VMEM((CH, 16), jnp.float32),
            pltpu.VMEM_SHARED((N, 16), jnp.float32),
            pltpu.SemaphoreType.DMA,
        ],
    )
    def deg(ei2_hbm, out_hbm, dstr_v, ones_v, acc_sh, ssem):
        c = lax.axis_index("c")
        s = lax.axis_index("s")
        wid = s * 2 + c
        cstart = wid * CPW + jnp.minimum(wid, XTRA)
        extra = wid < XTRA

        _zero_rows(ones_v, CH, 16)
        _zero_acc_slice(ones_v, acc_sh, s)
        pltpu.sync_copy(ei2_hbm.at[1].at[pl.ds(cstart, CPW)],
                        dstr_v.at[pl.ds(0, CPW)])

        @pl.when(extra)
        def _():
            pltpu.sync_copy(ei2_hbm.at[1].at[pl.ds(cstart + CPW, 1)],
                            dstr_v.at[pl.ds(CPW, 1)])

        one = jnp.ones((16,), jnp.float32)

        def fill(i, carry):
            ones_v[i, pl.ds(0, 16)] = one
            return carry

        lax.fori_loop(0, CH, fill, 0)
        plsc.subcore_barrier()

        DGRP = 13

        def body(k, carry):
            for t in range(DGRP):
                pltpu.make_async_copy(
                    ones_v, acc_sh.at[dstr_v.at[k * DGRP + t]],
                    ssem).start(add=True)
            for t in range(DGRP):
                pltpu.make_async_copy(
                    ones_v, acc_sh.at[dstr_v.at[k * DGRP + t]],
                    ssem).wait()
            return carry

        lax.fori_loop(0, CPW // DGRP, body, 0)

        @pl.when(extra)
        def _():
            pltpu.sync_copy(ones_v, acc_sh.at[dstr_v.at[CPW]], add=True)

        plsc.subcore_barrier()
        _copy_out(acc_sh, out_hbm, c, s)

    return deg


_R = 2000
_RP = _R // 8


def _dsel(dinvp):
    l = lax.broadcasted_iota(jnp.int32, (128, 8), 0)
    k = lax.broadcasted_iota(jnp.int32, (128, 8), 1)
    sel = jnp.where(l == 16 * k, 1.0, 0.0).astype(jnp.float32)
    return jnp.dot(dinvp, sel, preferred_element_type=jnp.float32,
                   precision=lax.Precision.HIGHEST)


def _rowscale(dsel, t):
    t3 = t.reshape(_RP, 8, t.shape[-1])
    return (dsel[:, :, None] * t3).reshape(t.shape)


def _tc1_body(degp, x, w1, g1, dinvp_out):
    r0 = pl.program_id(0) * _RP
    dinvp = lax.rsqrt(degp[0, pl.ds(r0, _RP), :]
                      + degp[1, pl.ds(r0, _RP), :] + 1.0)
    dinvp_out[pl.ds(r0, _RP), :] = dinvp
    u1 = jnp.dot(x[...], w1[...], preferred_element_type=jnp.float32)
    g1[...] = _rowscale(_dsel(dinvp), u1)


def _tc2_body(aggp, g1, dinvp, b1, w2, g2):
    r0 = pl.program_id(0) * _RP
    dsel = _dsel(dinvp[pl.ds(r0, _RP), :])
    h = _rowscale(dsel, aggp[0] + aggp[1] + g1[...]) + b1[...]
    h = jnp.maximum(h, 0.0)
    g2[...] = _rowscale(dsel, jnp.dot(h, w2[...],
                                      preferred_element_type=jnp.float32))


def _tc3_body(aggp, g2, dinvp, b2, out):
    r0 = pl.program_id(0) * _RP
    logits = _rowscale(_dsel(dinvp[pl.ds(r0, _RP), :]),
                       aggp[0] + aggp[1] + g2[...]) + b2[...]
    m = jnp.max(logits, axis=1, keepdims=True)
    e = logits - m
    out[...] = e - jnp.log(jnp.sum(jnp.exp(e), axis=1, keepdims=True))


def _row_spec(F):
    return pl.BlockSpec((_R, F), lambda i: (i, 0))


def _pair_spec(F):
    return pl.BlockSpec((2, _R, F), lambda i: (0, i, 0))


def _full_spec(a, b):
    return pl.BlockSpec((a, b), lambda i: (0, 0))


_packed_spec = pl.BlockSpec((N // 8, 128), lambda i: (0, 0))
_pairp_spec = pl.BlockSpec((2, N // 8, 128), lambda i: (0, 0, 0))

_tc1 = pl.pallas_call(
    _tc1_body,
    grid=(N // _R,),
    in_specs=[_pairp_spec, _row_spec(128), _full_spec(128, 128)],
    out_specs=[_row_spec(128), _packed_spec],
    out_shape=[jax.ShapeDtypeStruct((N, 128), jnp.float32),
               jax.ShapeDtypeStruct((N // 8, 128), jnp.float32)],
)

_tc2 = pl.pallas_call(
    _tc2_body,
    grid=(N // _R,),
    in_specs=[_pair_spec(128), _row_spec(128), _packed_spec,
              _full_spec(1, 128), _full_spec(128, 64)],
    out_specs=[_row_spec(64)],
    out_shape=[jax.ShapeDtypeStruct((N, 64), jnp.float32)],
)

_tc3 = pl.pallas_call(
    _tc3_body,
    grid=(N // _R,),
    in_specs=[_pair_spec(64), _row_spec(64), _packed_spec,
              _full_spec(1, 64)],
    out_specs=[_row_spec(64)],
    out_shape=[jax.ShapeDtypeStruct((N, 64), jnp.float32)],
)

_agg128 = _make_agg(128, 2)
_agg64 = _make_agg(64, 4)
_deg = _make_deg()


@jax.jit
def kernel(x, edge_index, W1, b1, W2, b2):
    ei2 = edge_index.reshape(2, NCHUNK, CH)
    degp = _deg(ei2).reshape(2, N // 8, 128)
    g1, dinvp = _tc1(degp, x, W1)
    aggp1 = _agg128(g1, ei2)
    (g2,) = _tc2(aggp1, g1, dinvp, b1.reshape(1, -1), W2)
    aggp2 = _agg64(g2, ei2)
    (out,) = _tc3(aggp2, g2, dinvp, b2.reshape(1, -1))
    return out
